# Initial kernel scaffold; baseline (speedup 1.0000x reference)
import jax
import jax.numpy as jnp
from jax.experimental import pallas as pl


def kernel(x_cheval, x_course, ei_participe, ei_rev_participe, lin_cheval_W, lin_cheval_b, lin_course_W, lin_course_b, Wl_part, bl_part, Wr_part, Wl_rev, bl_rev, Wr_rev, cls_W, cls_b):
    raise NotImplementedError("write your pallas kernel here")


# speedup vs baseline: 4.2939x; 4.2939x over previous
import functools

import jax
import jax.numpy as jnp
from jax import lax
from jax.experimental import pallas as pl
from jax.experimental.pallas import tpu as pltpu
from jax.experimental.pallas import tpu_sc as plsc

N = 10000
E = 320000
D = 128
H = 64
L = 3

NS = 16
NPAD = 10240
RPT = NPAD // NS
EPT = E // NS
B = 80
NCH = EPT // B

RB = 1000
GRID = N // RB


@functools.lru_cache(maxsize=None)
def _sc_kernels():
    mesh = plsc.VectorSubcoreMesh(core_axis_name="c", subcore_axis_name="s")
    f32 = jnp.float32

    def agg_body(hc_hbm, hr_hbm, srcp_hbm, dstp_hbm, srcv_hbm, dstv_hbm,
                 zeros_hbm, aggp_hbm, aggv_hbm, acc, sidx, didx, rows, sem):
        cid = lax.axis_index("c")
        sid = lax.axis_index("s")
        row0 = pl.multiple_of(sid * RPT, 8)
        pltpu.sync_copy(zeros_hbm, acc.at[pl.ds(row0, RPT)])
        plsc.subcore_barrier()

        def edge_loop(src_hbm, dst_hbm, table_hbm):
            def step(k, carry):
                off = pl.multiple_of(sid * EPT + k * B, 8)
                pltpu.sync_copy(src_hbm.at[pl.ds(off, B)], sidx)
                pltpu.sync_copy(dst_hbm.at[pl.ds(off, B)], didx)
                pltpu.async_copy(table_hbm.at[sidx], rows, sem).wait()
                pltpu.sync_copy(rows, acc.at[didx], add=True)
                return carry
            lax.fori_loop(0, NCH, step, 0)

        @pl.when(cid == 0)
        def _():
            edge_loop(srcp_hbm, dstp_hbm, hc_hbm)

        @pl.when(cid == 1)
        def _():
            edge_loop(srcv_hbm, dstv_hbm, hr_hbm)

        plsc.subcore_barrier()

        @pl.when(cid == 0)
        def _():
            pltpu.sync_copy(acc.at[pl.ds(row0, RPT)],
                            aggp_hbm.at[pl.ds(row0, RPT)])

        @pl.when(cid == 1)
        def _():
            pltpu.sync_copy(acc.at[pl.ds(row0, RPT)],
                            aggv_hbm.at[pl.ds(row0, RPT)])

    agg = pl.kernel(
        agg_body,
        mesh=mesh,
        compiler_params=pltpu.CompilerParams(use_tc_tiling_on_sc=False),
        out_type=[jax.ShapeDtypeStruct((NPAD, H), f32),
                  jax.ShapeDtypeStruct((NPAD, H), f32)],
        scratch_types=[
            pltpu.VMEM_SHARED((NPAD, H), f32),
            pltpu.VMEM((B,), jnp.int32),
            pltpu.VMEM((B,), jnp.int32),
            pltpu.VMEM((B, H), f32),
            pltpu.SemaphoreType.DMA,
        ],
    )

    def cnt_body(dstp_hbm, dstv_hbm, zeros_hbm, ones_hbm,
                 cntp_hbm, cntv_hbm, acc, didx, ones_v, sem):
        cid = lax.axis_index("c")
        sid = lax.axis_index("s")
        row0 = pl.multiple_of(sid * RPT, 8)
        pltpu.sync_copy(zeros_hbm, acc.at[pl.ds(row0, RPT)])
        pltpu.sync_copy(ones_hbm, ones_v)
        plsc.subcore_barrier()

        def edge_loop(dst_hbm):
            def step(k, carry):
                off = pl.multiple_of(sid * EPT + k * B, 8)
                pltpu.sync_copy(dst_hbm.at[pl.ds(off, B)], didx)
                pltpu.sync_copy(ones_v, acc.at[didx], add=True)
                return carry
            lax.fori_loop(0, NCH, step, 0)

        @pl.when(cid == 0)
        def _():
            edge_loop(dstp_hbm)

        @pl.when(cid == 1)
        def _():
            edge_loop(dstv_hbm)

        plsc.subcore_barrier()

        @pl.when(cid == 0)
        def _():
            pltpu.sync_copy(acc.at[pl.ds(row0, RPT)],
                            cntp_hbm.at[pl.ds(row0, RPT)])

        @pl.when(cid == 1)
        def _():
            pltpu.sync_copy(acc.at[pl.ds(row0, RPT)],
                            cntv_hbm.at[pl.ds(row0, RPT)])

    cnt = pl.kernel(
        cnt_body,
        mesh=mesh,
        compiler_params=pltpu.CompilerParams(use_tc_tiling_on_sc=False),
        out_type=[jax.ShapeDtypeStruct((NPAD, 16), f32),
                  jax.ShapeDtypeStruct((NPAD, 16), f32)],
        scratch_types=[
            pltpu.VMEM_SHARED((NPAD, 16), f32),
            pltpu.VMEM((B,), jnp.int32),
            pltpu.VMEM((B, 16), f32),
            pltpu.SemaphoreType.DMA,
        ],
    )

    return agg, cnt


def _proj_body(xc, xr, wc, bc, wr, br, hc, hr):
    hc[...] = jnp.maximum(
        jnp.dot(xc[...], wc[...], preferred_element_type=jnp.float32) + bc[...], 0.0)
    hr[...] = jnp.maximum(
        jnp.dot(xr[...], wr[...], preferred_element_type=jnp.float32) + br[...], 0.0)


def _proj(xc, xr, wc, bc, wr, br):
    f32 = jnp.float32
    row = pl.BlockSpec((RB, D), lambda i: (i, 0))
    w = pl.BlockSpec((D, H), lambda i: (0, 0))
    b = pl.BlockSpec((1, H), lambda i: (0, 0))
    out = pl.BlockSpec((RB, H), lambda i: (i, 0))
    return pl.pallas_call(
        _proj_body,
        grid=(GRID,),
        in_specs=[row, row, w, b, w, b],
        out_specs=[out, out],
        out_shape=[jax.ShapeDtypeStruct((N, H), f32),
                   jax.ShapeDtypeStruct((N, H), f32)],
    )(xc, xr, wc, bc.reshape(1, H), wr, br.reshape(1, H))


def _layer_body(hc, hr, aggp, aggv, cntp, cntv,
                wlp, blp, wrp, wlv, blv, wrv, nhc, nhr):
    inv_p = 1.0 / jnp.maximum(cntp[:, 0:1], 1.0)
    inv_v = 1.0 / jnp.maximum(cntv[:, 0:1], 1.0)
    new_r = (jnp.dot(aggp[...] * inv_p, wlp[...], preferred_element_type=jnp.float32)
             + blp[...]
             + jnp.dot(hr[...], wrp[...], preferred_element_type=jnp.float32))
    new_c = (jnp.dot(aggv[...] * inv_v, wlv[...], preferred_element_type=jnp.float32)
             + blv[...]
             + jnp.dot(hc[...], wrv[...], preferred_element_type=jnp.float32))
    nhc[...] = jnp.maximum(new_c, 0.0)
    nhr[...] = jnp.maximum(new_r, 0.0)


def _layer(hc, hr, aggp, aggv, cntp, cntv, wlp, blp, wrp, wlv, blv, wrv):
    f32 = jnp.float32
    row = pl.BlockSpec((RB, H), lambda i: (i, 0))
    cnt = pl.BlockSpec((RB, 16), lambda i: (i, 0))
    w = pl.BlockSpec((H, H), lambda i: (0, 0))
    b = pl.BlockSpec((1, H), lambda i: (0, 0))
    return pl.pallas_call(
        _layer_body,
        grid=(GRID,),
        in_specs=[row, row, row, row, cnt, cnt, w, b, w, w, b, w],
        out_specs=[row, row],
        out_shape=[jax.ShapeDtypeStruct((N, H), f32),
                   jax.ShapeDtypeStruct((N, H), f32)],
    )(hc, hr, aggp, aggv, cntp, cntv,
      wlp, blp.reshape(1, H), wrp, wlv, blv.reshape(1, H), wrv)


def _final_body(hc, aggv, cntv, wlv, blv, wrv, clsw, clsb, out):
    inv_v = 1.0 / jnp.maximum(cntv[:, 0:1], 1.0)
    new_c = (jnp.dot(aggv[...] * inv_v, wlv[...], preferred_element_type=jnp.float32)
             + blv[...]
             + jnp.dot(hc[...], wrv[...], preferred_element_type=jnp.float32))
    h = jnp.maximum(new_c, 0.0)
    out[...] = jnp.dot(h, clsw[...], preferred_element_type=jnp.float32) + clsb[...]


def _final(hc, aggv, cntv, wlv, blv, wrv, clsw_pad, clsb):
    f32 = jnp.float32
    row = pl.BlockSpec((RB, H), lambda i: (i, 0))
    cnt = pl.BlockSpec((RB, 16), lambda i: (i, 0))
    w = pl.BlockSpec((H, H), lambda i: (0, 0))
    b = pl.BlockSpec((1, H), lambda i: (0, 0))
    return pl.pallas_call(
        _final_body,
        grid=(GRID,),
        in_specs=[row, row, cnt, w, b, w,
                  pl.BlockSpec((H, 128), lambda i: (0, 0)),
                  pl.BlockSpec((1, 1), lambda i: (0, 0))],
        out_specs=pl.BlockSpec((RB, 128), lambda i: (i, 0)),
        out_shape=jax.ShapeDtypeStruct((N, 128), f32),
    )(hc, aggv, cntv, wlv, blv.reshape(1, H), wrv, clsw_pad,
      clsb.reshape(1, 1))


def kernel(x_cheval, x_course, ei_participe, ei_rev_participe,
           lin_cheval_W, lin_cheval_b, lin_course_W, lin_course_b,
           Wl_part, bl_part, Wr_part, Wl_rev, bl_rev, Wr_rev,
           cls_W, cls_b):
    f32 = jnp.float32
    agg_k, cnt_k = _sc_kernels()

    src_p, dst_p = ei_participe[0], ei_participe[1]
    src_v, dst_v = ei_rev_participe[0], ei_rev_participe[1]

    zeros64 = jnp.zeros((RPT, H), f32)
    zeros16 = jnp.zeros((RPT, 16), f32)
    ones16 = jnp.ones((B, 16), f32)
    clsw_pad = jnp.pad(cls_W, ((0, 0), (0, 128 - cls_W.shape[1])))

    h_c, h_r = _proj(x_cheval, x_course,
                     lin_cheval_W, lin_cheval_b, lin_course_W, lin_course_b)

    cntp, cntv = cnt_k(dst_p, dst_v, zeros16, ones16)
    cntp, cntv = cntp[:N], cntv[:N]

    out = None
    for l in range(L):
        aggp, aggv = agg_k(h_c, h_r, src_p, dst_p, src_v, dst_v, zeros64)
        aggp, aggv = aggp[:N], aggv[:N]
        if l < L - 1:
            h_c, h_r = _layer(h_c, h_r, aggp, aggv, cntp, cntv,
                              Wl_part[l], bl_part[l], Wr_part[l],
                              Wl_rev[l], bl_rev[l], Wr_rev[l])
        else:
            out = _final(h_c, aggv, cntv,
                         Wl_rev[l], bl_rev[l], Wr_rev[l], clsw_pad, cls_b)
    return out[:, :1]
